# Initial kernel scaffold; baseline (speedup 1.0000x reference)
import jax
import jax.numpy as jnp
from jax.experimental import pallas as pl


def kernel(seq, adj, W, b, prelu_a):
    raise NotImplementedError("write your pallas kernel here")


# speedup vs baseline: 16.4329x; 16.4329x over previous
import functools

import jax
import jax.numpy as jnp
from jax import lax
from jax.experimental import pallas as pl
from jax.experimental.pallas import tpu as pltpu
from jax.experimental.pallas import tpu_sc as plsc

N_NODES = 10000
D = 128
E = 320000
N_PAD = 10240
NUM_CORES = 2
NUM_SUBCORES = 16
NUM_TILES = NUM_CORES * NUM_SUBCORES
CHUNK = 128
EDGES_PER_TILE = 10112
NUM_CHUNKS = EDGES_PER_TILE // CHUNK
ROWS_PER_TILE = N_PAD // NUM_SUBCORES

_mesh = plsc.VectorSubcoreMesh(core_axis_name="c", subcore_axis_name="s")


@functools.partial(
    pl.kernel,
    out_type=jax.ShapeDtypeStruct((NUM_CORES, N_PAD), jnp.float32),
    mesh=_mesh,
    scratch_types=[
        pltpu.VMEM((CHUNK,), jnp.int32),
        pltpu.VMEM((CHUNK,), jnp.float32),
        pltpu.VMEM_SHARED((N_PAD,), jnp.float32),
    ],
)
def _deg_kernel(dst_hbm, zrow_hbm, deg_hbm, didx_v, ones_v, acc_sh):
    cid = lax.axis_index("c")
    sid = lax.axis_index("s")
    r0 = pl.multiple_of(sid * ROWS_PER_TILE, 8)
    pltpu.sync_copy(zrow_hbm, acc_sh.at[pl.ds(r0, ROWS_PER_TILE)])
    for i in range(CHUNK // 16):
        ones_v[pl.ds(i * 16, 16)] = jnp.ones((16,), jnp.float32)
    plsc.subcore_barrier()
    base = (cid * NUM_SUBCORES + sid) * EDGES_PER_TILE

    def body(c, carry):
        off = pl.multiple_of(base + c * CHUNK, 8)
        pltpu.sync_copy(dst_hbm.at[pl.ds(off, CHUNK)], didx_v)
        pltpu.sync_copy(ones_v, acc_sh.at[didx_v], add=True)
        return carry

    lax.fori_loop(0, NUM_CHUNKS, body, 0)
    plsc.subcore_barrier()
    pltpu.sync_copy(acc_sh.at[pl.ds(r0, ROWS_PER_TILE)],
                    deg_hbm.at[cid, pl.ds(r0, ROWS_PER_TILE)])


@functools.partial(
    pl.kernel,
    out_type=jax.ShapeDtypeStruct((NUM_CORES, N_PAD, D), jnp.float32),
    mesh=_mesh,
    scratch_types=[
        pltpu.VMEM((CHUNK,), jnp.int32),
        pltpu.VMEM((CHUNK,), jnp.int32),
        pltpu.VMEM((CHUNK, D), jnp.float32),
        pltpu.VMEM_SHARED((N_PAD, D), jnp.float32),
        pltpu.SemaphoreType.DMA,
    ],
)
def _edge_kernel(src_hbm, dst_hbm, y_hbm, zrows_hbm, acc_hbm,
                 sidx_v, didx_v, rows_v, acc_sh, sem):
    cid = lax.axis_index("c")
    sid = lax.axis_index("s")
    r0 = pl.multiple_of(sid * ROWS_PER_TILE, 8)
    pltpu.sync_copy(zrows_hbm, acc_sh.at[pl.ds(r0, ROWS_PER_TILE)])
    plsc.subcore_barrier()
    base = (cid * NUM_SUBCORES + sid) * EDGES_PER_TILE

    def body(c, carry):
        off = pl.multiple_of(base + c * CHUNK, 8)
        pltpu.sync_copy(src_hbm.at[pl.ds(off, CHUNK)], sidx_v)
        pltpu.sync_copy(dst_hbm.at[pl.ds(off, CHUNK)], didx_v)
        pltpu.async_copy(y_hbm.at[sidx_v], rows_v, sem).wait()
        pltpu.sync_copy(rows_v, acc_sh.at[didx_v], add=True)
        return carry

    lax.fori_loop(0, NUM_CHUNKS, body, 0)
    plsc.subcore_barrier()
    pltpu.sync_copy(acc_sh.at[pl.ds(r0, ROWS_PER_TILE)],
                    acc_hbm.at[cid, pl.ds(r0, ROWS_PER_TILE)])


_BLK = 2048


def _mm_body(x_ref, w_ref, degp_ref, y_ref, dinv_ref):
    deg = degp_ref[0, :] + degp_ref[1, :] + 1.0
    dinv = lax.rsqrt(deg)
    xw = jnp.dot(x_ref[...], w_ref[...], preferred_element_type=jnp.float32)
    y_ref[...] = xw * dinv[:, None]
    dinv_ref[...] = dinv


def _mm(x_p, W, degp):
    return pl.pallas_call(
        _mm_body,
        grid=(N_PAD // _BLK,),
        in_specs=[
            pl.BlockSpec((_BLK, D), lambda i: (i, 0)),
            pl.BlockSpec((D, D), lambda i: (0, 0)),
            pl.BlockSpec((NUM_CORES, _BLK), lambda i: (0, i)),
        ],
        out_specs=[
            pl.BlockSpec((_BLK, D), lambda i: (i, 0)),
            pl.BlockSpec((_BLK,), lambda i: (i,)),
        ],
        out_shape=[
            jax.ShapeDtypeStruct((N_PAD, D), jnp.float32),
            jax.ShapeDtypeStruct((N_PAD,), jnp.float32),
        ],
    )(x_p, W, degp)


def _final_body(accp_ref, y_ref, dinv_ref, b_ref, a_ref, out_ref):
    s = accp_ref[0] + accp_ref[1] + y_ref[...]
    h = s * dinv_ref[...][:, None] + b_ref[...][None, :]
    a = a_ref[0]
    out_ref[...] = jnp.where(h > 0, h, a * h)


def _final(accp, y, dinv, b, a):
    return pl.pallas_call(
        _final_body,
        grid=(N_PAD // _BLK,),
        in_specs=[
            pl.BlockSpec((NUM_CORES, _BLK, D), lambda i: (0, i, 0)),
            pl.BlockSpec((_BLK, D), lambda i: (i, 0)),
            pl.BlockSpec((_BLK,), lambda i: (i,)),
            pl.BlockSpec((D,), lambda i: (0,)),
            pl.BlockSpec(memory_space=pltpu.SMEM),
        ],
        out_specs=pl.BlockSpec((_BLK, D), lambda i: (i, 0)),
        out_shape=jax.ShapeDtypeStruct((N_PAD, D), jnp.float32),
    )(accp, y, dinv, b, a)


def kernel(seq, adj, W, b, prelu_a):
    src = adj[0].astype(jnp.int32)
    dst = adj[1].astype(jnp.int32)
    per_tile = E // NUM_TILES
    pad = jnp.full((NUM_TILES, EDGES_PER_TILE - per_tile), N_NODES, jnp.int32)
    src_p = jnp.concatenate([src.reshape(NUM_TILES, per_tile), pad], 1).reshape(-1)
    dst_p = jnp.concatenate([dst.reshape(NUM_TILES, per_tile), pad], 1).reshape(-1)
    x_p = jnp.pad(seq, ((0, N_PAD - N_NODES), (0, 0)))
    zrow = jnp.zeros((ROWS_PER_TILE,), jnp.float32)
    zrows = jnp.zeros((ROWS_PER_TILE, D), jnp.float32)

    degp = _deg_kernel(dst_p, zrow)
    y, dinv = _mm(x_p, W, degp)
    accp = _edge_kernel(src_p, dst_p, y, zrows)
    out = _final(accp, y, dinv, b, jnp.reshape(prelu_a, (1,)))
    return out[:N_NODES]
